# baseline (speedup 1.0000x reference)
import functools

import jax
import jax.numpy as jnp
from jax import lax
from jax.experimental import pallas as pl
from jax.experimental.pallas import tpu as pltpu
from jax.experimental.pallas import tpu_sc as plsc

_B = 512
_ROW = 4 * 64 * 64
_TABLE = 1000

_NC = 2
_NS = 16
_L = 16
_NW = _NC * _NS
_BPW = _B // _NW


def _sc_gather_body(t_hbm, sac_hbm, somac_hbm, a_hbm, b_hbm,
                    idx_v, sac_v, somac_v, a_v, b_v, sem_t, sem_a, sem_b):
    wid = lax.axis_index("s") * _NC + lax.axis_index("c")
    base = wid * _BPW
    ct = pltpu.async_copy(t_hbm.at[pl.ds(base, _BPW)], idx_v, sem_t)
    ca = pltpu.async_copy(sac_hbm, sac_v, sem_a)
    cb = pltpu.async_copy(somac_hbm, somac_v, sem_b)
    ct.wait()
    ca.wait()
    cb.wait()
    idx = idx_v[...]
    a_v[...] = plsc.load_gather(sac_v, [idx])
    b_v[...] = plsc.load_gather(somac_v, [idx])
    wa = pltpu.async_copy(a_v, a_hbm.at[0, pl.ds(base, _BPW)], sem_a)
    wb = pltpu.async_copy(b_v, b_hbm.at[0, pl.ds(base, _BPW)], sem_b)
    wa.wait()
    wb.wait()


@jax.jit
def _sc_gather(t, sac, somac):
    f = pl.kernel(
        _sc_gather_body,
        out_type=(
            jax.ShapeDtypeStruct((1, _B), jnp.float32),
            jax.ShapeDtypeStruct((1, _B), jnp.float32),
        ),
        mesh=plsc.VectorSubcoreMesh(core_axis_name="c", subcore_axis_name="s"),
        compiler_params=pltpu.CompilerParams(needs_layout_passes=False),
        scratch_types=[
            pltpu.VMEM((_BPW,), jnp.int32),
            pltpu.VMEM((_TABLE,), jnp.float32),
            pltpu.VMEM((_TABLE,), jnp.float32),
            pltpu.VMEM((_BPW,), jnp.float32),
            pltpu.VMEM((_BPW,), jnp.float32),
            pltpu.SemaphoreType.DMA,
            pltpu.SemaphoreType.DMA,
            pltpu.SemaphoreType.DMA,
        ],
    )
    return f(t, sac, somac)


def _tc_body(a_ref, b_ref, x_ref, n_ref, o_ref, n_out_ref):
    n = n_ref[...]
    o_ref[...] = a_ref[...] * x_ref[...] + b_ref[...] * n
    n_out_ref[...] = n


_ROWS_PER_BLOCK = 2048
_C, _H, _W = 4, 64, 64


@jax.jit
def _tc_scale_add(a, b, x, n):
    grid = (_ROW // _ROWS_PER_BLOCK,)
    blk = pl.BlockSpec((_ROWS_PER_BLOCK, _B), lambda i: (i, 0))
    cblk = pl.BlockSpec((1, _B), lambda i: (0, 0))
    return pl.pallas_call(
        _tc_body,
        grid=grid,
        in_specs=[cblk, cblk, blk, blk],
        out_specs=[blk, blk],
        out_shape=[
            jax.ShapeDtypeStruct((_ROW, _B), jnp.float32),
            jax.ShapeDtypeStruct((_ROW, _B), jnp.float32),
        ],
        compiler_params=pltpu.CompilerParams(
            dimension_semantics=("arbitrary",),
        ),
    )(a, b, x, n)


def kernel(x_0, noise, t, sqrt_alphas_cumprod, sqrt_one_minus_alphas_cumprod):
    t32 = t.astype(jnp.int32)
    a, b = _sc_gather(t32, sqrt_alphas_cumprod, sqrt_one_minus_alphas_cumprod)
    xt = jnp.transpose(x_0, (1, 2, 3, 0)).reshape(_ROW, _B)
    nt = jnp.transpose(noise, (1, 2, 3, 0)).reshape(_ROW, _B)
    out, n_out = _tc_scale_add(a, b, xt, nt)
    out = jnp.transpose(out.reshape(_C, _H, _W, _B), (3, 0, 1, 2))
    n_out = jnp.transpose(n_out.reshape(_C, _H, _W, _B), (3, 0, 1, 2))
    return (out, n_out)


# speedup vs baseline: 1.0294x; 1.0294x over previous
import functools

import jax
import jax.numpy as jnp
from jax import lax
from jax.experimental import pallas as pl
from jax.experimental.pallas import tpu as pltpu
from jax.experimental.pallas import tpu_sc as plsc

_B = 512
_ROW = 4 * 64 * 64
_TABLE = 1000

_NC = 1
_NS = 16
_L = 16
_NW = _NC * _NS
_BPW = _B // _NW


def _sc_gather_body(t_hbm, sac_hbm, somac_hbm, a_hbm, b_hbm,
                    idx_v, sac_v, somac_v, a_v, b_v, sem_t, sem_a, sem_b):
    wid = lax.axis_index("s") * _NC + lax.axis_index("c")
    base = wid * _BPW
    ct = pltpu.async_copy(t_hbm.at[pl.ds(base, _BPW)], idx_v, sem_t)
    ca = pltpu.async_copy(sac_hbm, sac_v, sem_a)
    cb = pltpu.async_copy(somac_hbm, somac_v, sem_b)
    ct.wait()
    ca.wait()
    cb.wait()
    for j in range(_BPW // _L):
        idx = idx_v[pl.ds(j * _L, _L)]
        a_v[pl.ds(j * _L, _L)] = plsc.load_gather(sac_v, [idx])
        b_v[pl.ds(j * _L, _L)] = plsc.load_gather(somac_v, [idx])
    wa = pltpu.async_copy(a_v, a_hbm.at[0, pl.ds(base, _BPW)], sem_a)
    wb = pltpu.async_copy(b_v, b_hbm.at[0, pl.ds(base, _BPW)], sem_b)
    wa.wait()
    wb.wait()


@jax.jit
def _sc_gather(t, sac, somac):
    f = pl.kernel(
        _sc_gather_body,
        out_type=(
            jax.ShapeDtypeStruct((1, _B), jnp.float32),
            jax.ShapeDtypeStruct((1, _B), jnp.float32),
        ),
        mesh=plsc.VectorSubcoreMesh(
            core_axis_name="c", subcore_axis_name="s", num_cores=_NC),
        compiler_params=pltpu.CompilerParams(needs_layout_passes=False),
        scratch_types=[
            pltpu.VMEM((_BPW,), jnp.int32),
            pltpu.VMEM((_TABLE,), jnp.float32),
            pltpu.VMEM((_TABLE,), jnp.float32),
            pltpu.VMEM((_BPW,), jnp.float32),
            pltpu.VMEM((_BPW,), jnp.float32),
            pltpu.SemaphoreType.DMA,
            pltpu.SemaphoreType.DMA,
            pltpu.SemaphoreType.DMA,
        ],
    )
    return f(t, sac, somac)


def _tc_body(a_ref, b_ref, x_ref, n_ref, o_ref, n_out_ref):
    n = n_ref[...]
    o_ref[...] = a_ref[...] * x_ref[...] + b_ref[...] * n
    n_out_ref[...] = n


_ROWS_PER_BLOCK = 2048
_C, _H, _W = 4, 64, 64


@jax.jit
def _tc_scale_add(a, b, x, n):
    grid = (_ROW // _ROWS_PER_BLOCK,)
    blk = pl.BlockSpec((_ROWS_PER_BLOCK, _B), lambda i: (i, 0))
    cblk = pl.BlockSpec((1, _B), lambda i: (0, 0))
    return pl.pallas_call(
        _tc_body,
        grid=grid,
        in_specs=[cblk, cblk, blk, blk],
        out_specs=[blk, blk],
        out_shape=[
            jax.ShapeDtypeStruct((_ROW, _B), jnp.float32),
            jax.ShapeDtypeStruct((_ROW, _B), jnp.float32),
        ],
        compiler_params=pltpu.CompilerParams(
            dimension_semantics=("arbitrary",),
        ),
    )(a, b, x, n)


def kernel(x_0, noise, t, sqrt_alphas_cumprod, sqrt_one_minus_alphas_cumprod):
    t32 = t.astype(jnp.int32)
    a, b = _sc_gather(t32, sqrt_alphas_cumprod, sqrt_one_minus_alphas_cumprod)
    xt = jnp.transpose(x_0, (1, 2, 3, 0)).reshape(_ROW, _B)
    nt = jnp.transpose(noise, (1, 2, 3, 0)).reshape(_ROW, _B)
    out, n_out = _tc_scale_add(a, b, xt, nt)
    out = jnp.transpose(out.reshape(_C, _H, _W, _B), (3, 0, 1, 2))
    n_out = jnp.transpose(n_out.reshape(_C, _H, _W, _B), (3, 0, 1, 2))
    return (out, n_out)
